# baseline (speedup 1.0000x reference)
import jax
import jax.numpy as jnp
from jax import lax
from jax.experimental import pallas as pl
from jax.experimental.pallas import tpu as pltpu
from jax.experimental.pallas import tpu_sc as plsc

N_ENT = 14541
N_REL = 237
DIM = 128
B = 16384
NC = 2
NS = 16
NW = NC * NS
BPW = B // NW
CH = 64
NCHUNK = BPW // CH
NK = DIM // 16


def _sc_body(h_hbm, t_hbm, r_hbm, relre_hbm, relim_hbm, ere_hbm, eim_hbm,
             out_hbm,
             hidx_v, tidx_v, r_v, rrel_v, rimv_v,
             hreA, himA, treA, timA, hreB, himB, treB, timB,
             acc_v, out_v, semA, semB):
    cid = lax.axis_index("c")
    sid = lax.axis_index("s")
    wid = sid * NC + cid
    base = wid * BPW

    pltpu.sync_copy(h_hbm.at[pl.ds(base, BPW)], hidx_v)
    pltpu.sync_copy(t_hbm.at[pl.ds(base, BPW)], tidx_v)
    pltpu.sync_copy(r_hbm, r_v)
    r = r_v[pl.ds(0, 16)][0]
    pltpu.sync_copy(relre_hbm.at[r], rrel_v)
    pltpu.sync_copy(relim_hbm.at[r], rimv_v)

    rre = [rrel_v[pl.ds(k * 16, 16)] for k in range(NK)]
    rim = [rimv_v[pl.ds(k * 16, 16)] for k in range(NK)]

    bufsA = (hreA, himA, treA, timA)
    bufsB = (hreB, himB, treB, timB)

    def fire(j, bufs, sem):
        idxh = hidx_v.at[pl.ds(j * CH, CH)]
        idxt = tidx_v.at[pl.ds(j * CH, CH)]
        pltpu.async_copy(ere_hbm.at[idxh], bufs[0], sem)
        pltpu.async_copy(eim_hbm.at[idxh], bufs[1], sem)
        pltpu.async_copy(ere_hbm.at[idxt], bufs[2], sem)
        pltpu.async_copy(eim_hbm.at[idxt], bufs[3], sem)

    def drain(bufs, sem):
        idx0 = hidx_v.at[pl.ds(0, CH)]
        for b in bufs:
            pltpu.make_async_copy(ere_hbm.at[idx0], b, sem).wait()

    def compute(j, bufs):
        hre_v, him_v, tre_v, tim_v = bufs

        @plsc.parallel_loop(0, CH, unroll=2)
        def _elem(e):
            acc = rre[0] * (hre_v[e, pl.ds(0, 16)] * tre_v[e, pl.ds(0, 16)]
                            + him_v[e, pl.ds(0, 16)] * tim_v[e, pl.ds(0, 16)])
            acc = acc + rim[0] * (hre_v[e, pl.ds(0, 16)] * tim_v[e, pl.ds(0, 16)]
                                  - him_v[e, pl.ds(0, 16)] * tre_v[e, pl.ds(0, 16)])
            for k in range(1, NK):
                hre = hre_v[e, pl.ds(k * 16, 16)]
                him = him_v[e, pl.ds(k * 16, 16)]
                tre = tre_v[e, pl.ds(k * 16, 16)]
                tim = tim_v[e, pl.ds(k * 16, 16)]
                acc = acc + (rre[k] * (hre * tre + him * tim)
                             + rim[k] * (hre * tim - him * tre))
            acc_v[pl.ds(e * 16, 16)] = acc

    lanes = lax.iota(jnp.int32, 16)
    cols = [lanes * 16 + ((lanes + s) & 15) for s in range(16)]

    def reduce_chunk(j):
        for g in range(CH // 16):
            gbase = g * 256
            v = plsc.load_gather(acc_v, [gbase + cols[0]])
            for s in range(1, 16):
                v = v + plsc.load_gather(acc_v, [gbase + cols[s]])
            out_v[pl.ds(j * CH + g * 16, 16)] = v

    fire(0, bufsA, semA)

    def pair_body(g, carry):
        j = 2 * g
        fire(j + 1, bufsB, semB)
        drain(bufsA, semA)
        compute(j, bufsA)
        reduce_chunk(j)

        @pl.when(g + 1 < NCHUNK // 2)
        def _():
            fire(j + 2, bufsA, semA)

        drain(bufsB, semB)
        compute(j + 1, bufsB)
        reduce_chunk(j + 1)
        return carry

    lax.fori_loop(0, NCHUNK // 2, pair_body, 0, unroll=False)
    pltpu.sync_copy(out_v, out_hbm.at[pl.ds(base, BPW)])


@jax.jit
def _complex_score(h, t, r_arr, rel_re, rel_im, ent_re, ent_im):
    mesh = plsc.VectorSubcoreMesh(
        core_axis_name="c", subcore_axis_name="s",
        num_cores=NC, num_subcores=NS)
    run = pl.kernel(
        _sc_body,
        out_type=jax.ShapeDtypeStruct((B,), jnp.float32),
        mesh=mesh,
        compiler_params=pltpu.CompilerParams(needs_layout_passes=False),
        scratch_types=[
            pltpu.VMEM((BPW,), jnp.int32),
            pltpu.VMEM((BPW,), jnp.int32),
            pltpu.VMEM((16,), jnp.int32),
            pltpu.VMEM((DIM,), jnp.float32),
            pltpu.VMEM((DIM,), jnp.float32),
            pltpu.VMEM((CH, DIM), jnp.float32),
            pltpu.VMEM((CH, DIM), jnp.float32),
            pltpu.VMEM((CH, DIM), jnp.float32),
            pltpu.VMEM((CH, DIM), jnp.float32),
            pltpu.VMEM((CH, DIM), jnp.float32),
            pltpu.VMEM((CH, DIM), jnp.float32),
            pltpu.VMEM((CH, DIM), jnp.float32),
            pltpu.VMEM((CH, DIM), jnp.float32),
            pltpu.VMEM((CH * 16,), jnp.float32),
            pltpu.VMEM((BPW,), jnp.float32),
            pltpu.SemaphoreType.DMA,
            pltpu.SemaphoreType.DMA,
        ],
    )
    return run(h, t, r_arr, rel_re, rel_im, ent_re, ent_im)


def kernel(predict_h, predict_t, predict_r, ent_re, ent_im, rel_re, rel_im):
    h = predict_h.astype(jnp.int32)
    t = predict_t.astype(jnp.int32)
    r_arr = jnp.full((16,), predict_r, dtype=jnp.int32)
    return _complex_score(h, t, r_arr, rel_re, rel_im, ent_re, ent_im)


# speedup vs baseline: 1.0522x; 1.0522x over previous
import jax
import jax.numpy as jnp
from jax import lax
from jax.experimental import pallas as pl
from jax.experimental.pallas import tpu as pltpu
from jax.experimental.pallas import tpu_sc as plsc

N_ENT = 14541
N_REL = 237
DIM = 128
B = 16384
NC = 2
NS = 16
NW = NC * NS
BPW = B // NW
CH = 64
NCHUNK = BPW // CH
NK = DIM // 16


def _sc_body(h_hbm, t_hbm, r_hbm, relre_hbm, relim_hbm, ere_hbm, eim_hbm,
             out_hbm,
             hidx_v, tidx_v, r_v, rrel_v, rimv_v,
             hreA, himA, treA, timA, hreB, himB, treB, timB,
             acc_v, out_v, semA, semB):
    cid = lax.axis_index("c")
    sid = lax.axis_index("s")
    wid = sid * NC + cid
    base = wid * BPW

    ic1 = pltpu.async_copy(h_hbm.at[pl.ds(base, BPW)], hidx_v, semB)
    ic2 = pltpu.async_copy(t_hbm.at[pl.ds(base, BPW)], tidx_v, semB)
    pltpu.sync_copy(r_hbm, r_v)
    r = r_v[pl.ds(0, 16)][0]
    pltpu.sync_copy(relre_hbm.at[r], rrel_v)
    pltpu.sync_copy(relim_hbm.at[r], rimv_v)
    ic1.wait()
    ic2.wait()

    rre = [rrel_v[pl.ds(k * 16, 16)] for k in range(NK)]
    rim = [rimv_v[pl.ds(k * 16, 16)] for k in range(NK)]

    bufsA = (hreA, himA, treA, timA)
    bufsB = (hreB, himB, treB, timB)

    def fire(j, bufs, sem):
        idxh = hidx_v.at[pl.ds(j * CH, CH)]
        idxt = tidx_v.at[pl.ds(j * CH, CH)]
        pltpu.async_copy(ere_hbm.at[idxh], bufs[0], sem)
        pltpu.async_copy(eim_hbm.at[idxh], bufs[1], sem)
        pltpu.async_copy(ere_hbm.at[idxt], bufs[2], sem)
        pltpu.async_copy(eim_hbm.at[idxt], bufs[3], sem)

    def drain(bufs, sem):
        idx0 = hidx_v.at[pl.ds(0, CH)]
        for b in bufs:
            pltpu.make_async_copy(ere_hbm.at[idx0], b, sem).wait()

    def compute(j, bufs):
        hre_v, him_v, tre_v, tim_v = bufs

        @plsc.parallel_loop(0, CH, unroll=2)
        def _elem(e):
            acc = rre[0] * (hre_v[e, pl.ds(0, 16)] * tre_v[e, pl.ds(0, 16)]
                            + him_v[e, pl.ds(0, 16)] * tim_v[e, pl.ds(0, 16)])
            acc = acc + rim[0] * (hre_v[e, pl.ds(0, 16)] * tim_v[e, pl.ds(0, 16)]
                                  - him_v[e, pl.ds(0, 16)] * tre_v[e, pl.ds(0, 16)])
            for k in range(1, NK):
                hre = hre_v[e, pl.ds(k * 16, 16)]
                him = him_v[e, pl.ds(k * 16, 16)]
                tre = tre_v[e, pl.ds(k * 16, 16)]
                tim = tim_v[e, pl.ds(k * 16, 16)]
                acc = acc + (rre[k] * (hre * tre + him * tim)
                             + rim[k] * (hre * tim - him * tre))
            acc_v[pl.ds((j * CH + e) * 16, 16)] = acc

    fire(0, bufsA, semA)

    def pair_body(g, carry):
        j = 2 * g
        fire(j + 1, bufsB, semB)
        drain(bufsA, semA)
        compute(j, bufsA)

        @pl.when(g + 1 < NCHUNK // 2)
        def _():
            fire(j + 2, bufsA, semA)

        drain(bufsB, semB)
        compute(j + 1, bufsB)
        return carry

    lax.fori_loop(0, NCHUNK // 2, pair_body, 0, unroll=False)

    lanes = lax.iota(jnp.int32, 16)
    cols = [lanes * 16 + ((lanes + s) & 15) for s in range(16)]

    def group_body(g, carry):
        gbase = g * 256
        v = plsc.load_gather(acc_v, [gbase + cols[0]])
        for s in range(1, 16):
            v = v + plsc.load_gather(acc_v, [gbase + cols[s]])
        out_v[pl.ds(g * 16, 16)] = v
        return carry

    lax.fori_loop(0, BPW // 16, group_body, 0, unroll=False)
    pltpu.sync_copy(out_v, out_hbm.at[pl.ds(base, BPW)])


@jax.jit
def _complex_score(h, t, r_arr, rel_re, rel_im, ent_re, ent_im):
    mesh = plsc.VectorSubcoreMesh(
        core_axis_name="c", subcore_axis_name="s",
        num_cores=NC, num_subcores=NS)
    run = pl.kernel(
        _sc_body,
        out_type=jax.ShapeDtypeStruct((B,), jnp.float32),
        mesh=mesh,
        compiler_params=pltpu.CompilerParams(needs_layout_passes=False),
        scratch_types=[
            pltpu.VMEM((BPW,), jnp.int32),
            pltpu.VMEM((BPW,), jnp.int32),
            pltpu.VMEM((16,), jnp.int32),
            pltpu.VMEM((DIM,), jnp.float32),
            pltpu.VMEM((DIM,), jnp.float32),
            pltpu.VMEM((CH, DIM), jnp.float32),
            pltpu.VMEM((CH, DIM), jnp.float32),
            pltpu.VMEM((CH, DIM), jnp.float32),
            pltpu.VMEM((CH, DIM), jnp.float32),
            pltpu.VMEM((CH, DIM), jnp.float32),
            pltpu.VMEM((CH, DIM), jnp.float32),
            pltpu.VMEM((CH, DIM), jnp.float32),
            pltpu.VMEM((CH, DIM), jnp.float32),
            pltpu.VMEM((BPW * 16,), jnp.float32),
            pltpu.VMEM((BPW,), jnp.float32),
            pltpu.SemaphoreType.DMA,
            pltpu.SemaphoreType.DMA,
        ],
    )
    return run(h, t, r_arr, rel_re, rel_im, ent_re, ent_im)


def kernel(predict_h, predict_t, predict_r, ent_re, ent_im, rel_re, rel_im):
    h = predict_h.astype(jnp.int32)
    t = predict_t.astype(jnp.int32)
    r_arr = jnp.full((16,), predict_r, dtype=jnp.int32)
    return _complex_score(h, t, r_arr, rel_re, rel_im, ent_re, ent_im)


# speedup vs baseline: 1.0853x; 1.0314x over previous
import jax
import jax.numpy as jnp
from jax import lax
from jax.experimental import pallas as pl
from jax.experimental.pallas import tpu as pltpu
from jax.experimental.pallas import tpu_sc as plsc

N_ENT = 14541
N_REL = 237
DIM = 128
B = 16384
NC = 2
NS = 16
NW = NC * NS
BPW = B // NW
CH = 32
NCHUNK = BPW // CH
NK = DIM // 16


def _sc_body(h_hbm, t_hbm, r_hbm, relre_hbm, relim_hbm, ere_hbm, eim_hbm,
             out_hbm,
             hidx_v, tidx_v, r_v, rrel_v, rimv_v,
             hreA, himA, treA, timA, hreB, himB, treB, timB,
             hreC, himC, treC, timC, hreD, himD, treD, timD,
             acc_v, out_v, semA, semB, semC, semD):
    cid = lax.axis_index("c")
    sid = lax.axis_index("s")
    wid = sid * NC + cid
    base = wid * BPW

    ic1 = pltpu.async_copy(h_hbm.at[pl.ds(base, BPW)], hidx_v, semB)
    ic2 = pltpu.async_copy(t_hbm.at[pl.ds(base, BPW)], tidx_v, semB)
    pltpu.sync_copy(r_hbm, r_v)
    r = r_v[pl.ds(0, 16)][0]
    pltpu.sync_copy(relre_hbm.at[r], rrel_v)
    pltpu.sync_copy(relim_hbm.at[r], rimv_v)
    ic1.wait()
    ic2.wait()

    rre = [rrel_v[pl.ds(k * 16, 16)] for k in range(NK)]
    rim = [rimv_v[pl.ds(k * 16, 16)] for k in range(NK)]

    sets = [(hreA, himA, treA, timA), (hreB, himB, treB, timB),
            (hreC, himC, treC, timC), (hreD, himD, treD, timD)]
    sems = [semA, semB, semC, semD]

    def fire(j, bufs, sem):
        idxh = hidx_v.at[pl.ds(j * CH, CH)]
        idxt = tidx_v.at[pl.ds(j * CH, CH)]
        pltpu.async_copy(ere_hbm.at[idxh], bufs[0], sem)
        pltpu.async_copy(eim_hbm.at[idxh], bufs[1], sem)
        pltpu.async_copy(ere_hbm.at[idxt], bufs[2], sem)
        pltpu.async_copy(eim_hbm.at[idxt], bufs[3], sem)

    def drain(bufs, sem):
        idx0 = hidx_v.at[pl.ds(0, CH)]
        for b in bufs:
            pltpu.make_async_copy(ere_hbm.at[idx0], b, sem).wait()

    def compute(j, bufs):
        hre_v, him_v, tre_v, tim_v = bufs

        @plsc.parallel_loop(0, CH, unroll=2)
        def _elem(e):
            acc = rre[0] * (hre_v[e, pl.ds(0, 16)] * tre_v[e, pl.ds(0, 16)]
                            + him_v[e, pl.ds(0, 16)] * tim_v[e, pl.ds(0, 16)])
            acc = acc + rim[0] * (hre_v[e, pl.ds(0, 16)] * tim_v[e, pl.ds(0, 16)]
                                  - him_v[e, pl.ds(0, 16)] * tre_v[e, pl.ds(0, 16)])
            for k in range(1, NK):
                hre = hre_v[e, pl.ds(k * 16, 16)]
                him = him_v[e, pl.ds(k * 16, 16)]
                tre = tre_v[e, pl.ds(k * 16, 16)]
                tim = tim_v[e, pl.ds(k * 16, 16)]
                acc = acc + (rre[k] * (hre * tre + him * tim)
                             + rim[k] * (hre * tim - him * tre))
            acc_v[pl.ds((j * CH + e) * 16, 16)] = acc

    fire(0, sets[0], sems[0])
    fire(1, sets[1], sems[1])
    fire(2, sets[2], sems[2])

    def quad_body(g, carry):
        j = 4 * g
        for p in range(4):
            c = j + p
            nxt = c + 3

            @pl.when(nxt < NCHUNK)
            def _(nxt=nxt, p=p):
                fire(nxt, sets[(p + 3) % 4], sems[(p + 3) % 4])

            drain(sets[p], sems[p])
            compute(c, sets[p])
        return carry

    lax.fori_loop(0, NCHUNK // 4, quad_body, 0, unroll=False)

    lanes = lax.iota(jnp.int32, 16)
    cols = [lanes * 16 + ((lanes + s) & 15) for s in range(16)]

    def group_body(g, carry):
        gbase = g * 256
        v = plsc.load_gather(acc_v, [gbase + cols[0]])
        for s in range(1, 16):
            v = v + plsc.load_gather(acc_v, [gbase + cols[s]])
        out_v[pl.ds(g * 16, 16)] = v
        return carry

    lax.fori_loop(0, BPW // 16, group_body, 0, unroll=False)
    pltpu.sync_copy(out_v, out_hbm.at[pl.ds(base, BPW)])


@jax.jit
def _complex_score(h, t, r_arr, rel_re, rel_im, ent_re, ent_im):
    mesh = plsc.VectorSubcoreMesh(
        core_axis_name="c", subcore_axis_name="s",
        num_cores=NC, num_subcores=NS)
    run = pl.kernel(
        _sc_body,
        out_type=jax.ShapeDtypeStruct((B,), jnp.float32),
        mesh=mesh,
        compiler_params=pltpu.CompilerParams(needs_layout_passes=False),
        scratch_types=[
            pltpu.VMEM((BPW,), jnp.int32),
            pltpu.VMEM((BPW,), jnp.int32),
            pltpu.VMEM((16,), jnp.int32),
            pltpu.VMEM((DIM,), jnp.float32),
            pltpu.VMEM((DIM,), jnp.float32),
        ] + [pltpu.VMEM((CH, DIM), jnp.float32)] * 16
          + [pltpu.VMEM((BPW * 16,), jnp.float32),
             pltpu.VMEM((BPW,), jnp.float32),
             pltpu.SemaphoreType.DMA,
             pltpu.SemaphoreType.DMA,
             pltpu.SemaphoreType.DMA,
             pltpu.SemaphoreType.DMA,
        ],
    )
    return run(h, t, r_arr, rel_re, rel_im, ent_re, ent_im)


def kernel(predict_h, predict_t, predict_r, ent_re, ent_im, rel_re, rel_im):
    h = predict_h.astype(jnp.int32)
    t = predict_t.astype(jnp.int32)
    r_arr = jnp.full((16,), predict_r, dtype=jnp.int32)
    return _complex_score(h, t, r_arr, rel_re, rel_im, ent_re, ent_im)
